# Initial kernel scaffold; baseline (speedup 1.0000x reference)
import jax
import jax.numpy as jnp
from jax.experimental import pallas as pl


def kernel(x, Wg, w1, w2):
    raise NotImplementedError("write your pallas kernel here")


# speedup vs baseline: 1.5551x; 1.5551x over previous
import functools
import jax
import jax.numpy as jnp
from jax.experimental import pallas as pl
from jax.experimental.pallas import tpu as pltpu

T, D, F, E = 2048, 1024, 2048, 8
NF = 2
FB = F // NF


def _dense_body(x_ref, wg_ref, w1_ref, w2_ref, o_ref, acc, comb):
    e = pl.program_id(0)
    f = pl.program_id(1)
    step = e * NF + f

    @pl.when(step == 0)
    def _():
        logits = x_ref[...] @ wg_ref[...]
        p = jax.nn.softmax(logits, axis=-1)
        iota = jax.lax.broadcasted_iota(jnp.int32, (T, E), 1)
        m1 = jnp.max(p, axis=-1, keepdims=True)
        i1 = jnp.min(jnp.where(p == m1, iota, E), axis=-1, keepdims=True)
        sel1 = iota == i1
        p2 = jnp.where(sel1, -jnp.inf, p)
        m2 = jnp.max(p2, axis=-1, keepdims=True)
        i2 = jnp.min(jnp.where(p2 == m2, iota, E), axis=-1, keepdims=True)
        sel2 = iota == i2
        s = m1 + m2
        comb[...] = jnp.where(sel1, m1 / s, jnp.where(sel2, m2 / s, 0.0))

    h = jnp.maximum(x_ref[...] @ w1_ref[0], 0.0)
    y = h @ w2_ref[0]
    iota = jax.lax.broadcasted_iota(jnp.int32, (T, E), 1)
    c = jnp.sum(jnp.where(iota == e, comb[...], 0.0), axis=-1, keepdims=True)

    @pl.when(step == 0)
    def _():
        acc[...] = c * y

    @pl.when(step > 0)
    def _():
        acc[...] += c * y

    @pl.when(step == E * NF - 1)
    def _():
        o_ref[...] = acc[...]


def kernel(x, Wg, w1, w2):
    return pl.pallas_call(
        _dense_body,
        grid=(E, NF),
        in_specs=[
            pl.BlockSpec((T, D), lambda e, f: (0, 0)),
            pl.BlockSpec((D, E), lambda e, f: (0, 0)),
            pl.BlockSpec((1, D, FB), lambda e, f: (e, 0, f)),
            pl.BlockSpec((1, FB, D), lambda e, f: (e, f, 0)),
        ],
        out_specs=pl.BlockSpec((T, D), lambda e, f: (0, 0)),
        out_shape=jax.ShapeDtypeStruct((T, D), jnp.float32),
        scratch_shapes=[
            pltpu.VMEM((T, D), jnp.float32),
            pltpu.VMEM((T, E), jnp.float32),
        ],
        compiler_params=pltpu.CompilerParams(
            dimension_semantics=("arbitrary", "arbitrary"),
        ),
    )(x, Wg, w1, w2)
